# baseline (speedup 1.0000x reference)
import jax
import jax.numpy as jnp
from jax import lax
from jax.experimental import pallas as pl
from jax.experimental.pallas import tpu as pltpu
from jax.experimental.pallas import tpu_sc as plsc

_NUM_USERS = 30000
_NUM_ITEMS = 70000
_N = _NUM_USERS + _NUM_ITEMS
_DIM = 32
_H = _DIM // 2
_E = 1600000
_HOP = 3
_BATCH = 4096

_NSUB = 16
_BE = 512
_NB = 200
_NBA = _NB + 2
_EPW = _BE * _NB
_E_ALLOC = _BE * _NBA * _NSUB
_NT = _NB // 2
_N_PAD = 100352
_RPW = _N_PAD // _NSUB
_ZCH = 224
_FCH = 448
_UPW = _BATCH // _NSUB


def _body(x0, comb, uids, iids,
          x1, x2, x3, out_u, out_i,
          acc, comb0, comb1, rows0, rows1, zbuf, uid_v,
          gsem0, gsem1, lsem0, lsem1):
    c = lax.axis_index("c")
    s = lax.axis_index("s")
    coff = c * _N_PAD
    combs = (comb0, comb1)
    rows = (rows0, rows1)
    gsems = (gsem0, gsem1)
    lsems = (lsem0, lsem1)
    meta_base = (c * _NSUB + s) * _NBA * 3

    def _zb(i, carry):
        zbuf[i, :] = jnp.zeros((_H,), jnp.float32)
        return carry
    lax.fori_loop(0, _ZCH, _zb, 0)

    def _wait_gather(p):
        pltpu.make_async_copy(x0.at[pl.ds(0, _BE)], rows[p], gsems[p]).wait()

    def _wait_meta(p):
        pltpu.make_async_copy(comb.at[pl.ds(0, 3)], combs[p], lsems[p]).wait()

    tabs = [x0, x1, x2, x3]
    for h in range(_HOP):
        xin, xout = tabs[h], tabs[h + 1]

        ns_zero = jax.named_scope("ph_zero"); ns_zero.__enter__()
        zd = [pltpu.async_copy(zbuf,
                               acc.at[pl.ds(s * _RPW + k * _ZCH, _ZCH)],
                               lsem0)
              for k in range(_RPW // _ZCH)]
        for p in (0, 1):
            pltpu.sync_copy(comb.at[pl.ds(meta_base + p * 3, 3)], combs[p])
            pltpu.async_copy(xin.at[combs[p].at[0]], rows[p], gsems[p])
        for d in zd:
            d.wait()
        plsc.subcore_barrier()
        ns_zero.__exit__(None, None, None)

        ns_edges = jax.named_scope("ph_edges"); ns_edges.__enter__()
        def _blk(t, carry):
            for p in (0, 1):
                b = 2 * t + p
                _wait_gather(p)

                def _mul(q, carry2):
                    e0 = q * 16
                    v16 = plsc.bitcast(combs[p][2, pl.ds(e0, 16)],
                                       jnp.float32)
                    for u in range(16):
                        e = e0 + u
                        rows[p][e, :] = rows[p][e, :] * v16[u]
                    return carry2
                lax.fori_loop(0, _BE // 16, _mul, 0)

                pltpu.sync_copy(rows[p], acc.at[combs[p].at[1]], add=True)
                pltpu.async_copy(comb.at[pl.ds(meta_base + (b + 2) * 3, 3)],
                                 combs[p], lsems[p])
            for p in (0, 1):
                _wait_meta(p)
                pltpu.async_copy(xin.at[combs[p].at[0]], rows[p], gsems[p])
            return carry
        lax.fori_loop(0, _NT, _blk, 0)
        for p in (0, 1):
            _wait_gather(p)
        plsc.subcore_barrier()
        ns_edges.__exit__(None, None, None)

        ns_flush = jax.named_scope("ph_flush"); ns_flush.__enter__()
        fd = [None, None]
        for k in range(_RPW // _FCH):
            p = k % 2
            if fd[p] is not None:
                fd[p].wait()
            r0 = s * _RPW + k * _FCH
            pltpu.sync_copy(acc.at[pl.ds(r0, _FCH)],
                            rows[p].at[pl.ds(0, _FCH)])
            fd[p] = pltpu.async_copy(rows[p].at[pl.ds(0, _FCH)],
                                     xout.at[pl.ds(coff + r0, _FCH)],
                                     gsems[p])
        for d in fd:
            d.wait()
        plsc.subcore_barrier()
        ns_flush.__exit__(None, None, None)

    def _emit(ids, tab_off, out_ref):
        pltpu.sync_copy(ids.at[pl.ds(s * _UPW, _UPW)], uid_v)
        for j in range(_UPW // 16):
            sl = pl.ds(j * 16, 16)
            uid_v[sl] = uid_v[sl] + tab_off
        dsts = ((rows0, 0), (rows0, _UPW), (rows1, 0), (rows1, _UPW))
        gd = [pltpu.async_copy(xt.at[uid_v], r.at[pl.ds(o, _UPW)], gsem0)
              for xt, (r, o) in zip((x0, x1, x2, x3), dsts)]
        for d in gd:
            d.wait()

        def _avg(q, carry2):
            e0 = q * 8
            for u in range(8):
                e = e0 + u
                v = ((rows0[e, :] + rows0[_UPW + e, :])
                     + (rows1[e, :] + rows1[_UPW + e, :]))
                rows0[e, :] = v * jnp.float32(0.25)
            return carry2
        lax.fori_loop(0, _UPW // 8, _avg, 0)
        pltpu.sync_copy(rows0.at[pl.ds(0, _UPW)],
                        out_ref.at[pl.ds(c * _BATCH + s * _UPW, _UPW)])

    _emit(uids, coff, out_u)
    _emit(iids, coff + _NUM_USERS, out_i)


@jax.jit
def _run(x0, comb, uids, iids):
    f32, i32 = jnp.float32, jnp.int32
    call = pl.kernel(
        _body,
        out_type=[
            jax.ShapeDtypeStruct((2 * _N_PAD, _H), f32),
            jax.ShapeDtypeStruct((2 * _N_PAD, _H), f32),
            jax.ShapeDtypeStruct((2 * _N_PAD, _H), f32),
            jax.ShapeDtypeStruct((2 * _BATCH, _H), f32),
            jax.ShapeDtypeStruct((2 * _BATCH, _H), f32),
        ],
        mesh=plsc.VectorSubcoreMesh(core_axis_name="c", subcore_axis_name="s"),
        scratch_types=[
            pltpu.VMEM_SHARED((_N_PAD, _H), f32),
            pltpu.VMEM((3, _BE), i32),
            pltpu.VMEM((3, _BE), i32),
            pltpu.VMEM((_BE, _H), f32),
            pltpu.VMEM((_BE, _H), f32),
            pltpu.VMEM((_ZCH, _H), f32),
            pltpu.VMEM((_UPW,), i32),
            pltpu.SemaphoreType.DMA,
            pltpu.SemaphoreType.DMA,
            pltpu.SemaphoreType.DMA,
            pltpu.SemaphoreType.DMA,
        ],
        compiler_params=pltpu.CompilerParams(use_tc_tiling_on_sc=False,
                                             needs_layout_passes=False),
        name="light_gcn_sc",
    )
    return call(x0, comb, uids, iids)


def kernel(user_emb, item_emb, edge_vals, edge_index, user_ids, item_ids):
    f32, i32 = jnp.float32, jnp.int32
    rpad = _N_PAD - _N
    lo = jnp.concatenate(
        [user_emb[:, :_H], item_emb[:, :_H], jnp.zeros((rpad, _H), f32)], axis=0)
    hi = jnp.concatenate(
        [user_emb[:, _H:], item_emb[:, _H:], jnp.zeros((rpad, _H), f32)], axis=0)
    x0 = jnp.concatenate([lo, hi], axis=0)

    epad = _EPW * _NSUB - _E
    def _blocks(flat):
        b3 = flat.reshape(_NSUB, _NB, _BE)
        return jnp.pad(b3, ((0, 0), (0, _NBA - _NB), (0, 0))).reshape(-1, _BE)
    src_b = _blocks(jnp.concatenate([edge_index[0], jnp.zeros((epad,), i32)]))
    dst_b = _blocks(jnp.concatenate([edge_index[1], jnp.zeros((epad,), i32)]))
    val_b = _blocks(jax.lax.bitcast_convert_type(
        jnp.concatenate([edge_vals, jnp.zeros((epad,), f32)]), i32))
    comb = jnp.concatenate(
        [jnp.stack([src_b + cc * _N_PAD, dst_b, val_b], axis=1)
         for cc in (0, 1)], axis=0).reshape(-1, _BE)

    _, _, _, ou, oi = _run(x0, comb, user_ids, item_ids)
    users = jnp.concatenate([ou[:_BATCH], ou[_BATCH:]], axis=1)
    items = jnp.concatenate([oi[:_BATCH], oi[_BATCH:]], axis=1)
    return users, items


# speedup vs baseline: 1.3704x; 1.3704x over previous
import jax
import jax.numpy as jnp
from jax import lax
from jax.experimental import pallas as pl
from jax.experimental.pallas import tpu as pltpu
from jax.experimental.pallas import tpu_sc as plsc

_NUM_USERS = 30000
_NUM_ITEMS = 70000
_N = _NUM_USERS + _NUM_ITEMS
_DIM = 32
_H = _DIM // 2
_E = 1600000
_HOP = 3
_BATCH = 4096

_NSUB = 16
_BE = 512
_NB = 200
_EPW = _BE * _NB
_NT = _NB // 2
_N_PAD = 100352
_RPW = _N_PAD // _NSUB
_ZCH = 224
_FCH = 448
_UPW = _BATCH // _NSUB


def _body(emb2, comb, uids, iids,
          x1, x2, x3, out_u, out_i,
          acc, ma0, mb0, ma1, mb1, rows0, rows1, zbuf, uidv, uid0v,
          gsem0, gsem1, lsem0, lsem1, zsem):
    c = lax.axis_index("c")
    s = lax.axis_index("s")
    coff = c * _N_PAD
    rows = (rows0, rows1)
    gsems = (gsem0, gsem1)
    lsems = (lsem0, lsem1)
    metas = ((ma0, mb0), (ma1, mb1))

    def _zb(i, carry):
        zbuf[i, :] = jnp.zeros((_H,), jnp.float32)
        return carry
    lax.fori_loop(0, _ZCH, _zb, 0)

    def _meta_load(bidx, mref, sem):
        r0 = (s * _NB + bidx) * 3
        return pltpu.async_copy(comb.at[pl.ds(r0, 3)], mref, sem)

    def _xform(mref, h):
        for j in range(_BE // 16):
            sl = pl.ds(j * 16, 16)
            if h == 0:
                mref[0, sl] = mref[0, sl] * 2 + c
            else:
                mref[0, sl] = mref[0, sl] + coff

    def _wait_gather(p):
        pltpu.make_async_copy(emb2.at[pl.ds(0, _BE)], rows[p],
                              gsems[p]).wait()

    def _wait_meta(p):
        pltpu.make_async_copy(comb.at[pl.ds(0, 3)], metas[p][0],
                              lsems[p]).wait()

    tabs = [emb2, x1, x2, x3]
    for h in range(_HOP):
        xin, xout = tabs[h], tabs[h + 1]

        zd = [pltpu.async_copy(zbuf,
                               acc.at[pl.ds(s * _RPW + k * _ZCH, _ZCH)],
                               zsem)
              for k in range(_RPW // _ZCH)]
        for p in (0, 1):
            _meta_load(p, metas[p][0], lsems[p]).wait()
            _xform(metas[p][0], h)
            pltpu.async_copy(xin.at[metas[p][0].at[0]], rows[p], gsems[p])
            _meta_load(p + 2, metas[p][1], lsems[p])
        for d in zd:
            d.wait()
        plsc.subcore_barrier()

        def _blk(t2, carry):
            for q in (0, 1):
                t = 2 * t2 + q
                for p in (0, 1):
                    b = 2 * t + p
                    mx, my = metas[p][q], metas[p][1 - q]
                    _wait_gather(p)

                    def _mul(i, carry2):
                        e0 = i * 16
                        v16 = plsc.bitcast(mx[2, pl.ds(e0, 16)], jnp.float32)
                        for u in range(16):
                            e = e0 + u
                            rows[p][e, :] = rows[p][e, :] * v16[u]
                        return carry2
                    lax.fori_loop(0, _BE // 16, _mul, 0)

                    pltpu.sync_copy(rows[p], acc.at[mx.at[1]], add=True)
                    _wait_meta(p)
                    _meta_load(jnp.minimum(b + 4, _NB - 1), mx, lsems[p])
                    _xform(my, h)
                    pltpu.async_copy(xin.at[my.at[0]], rows[p], gsems[p])
            return carry
        lax.fori_loop(0, _NT // 2, _blk, 0)
        for p in (0, 1):
            _wait_gather(p)
            _wait_meta(p)
        plsc.subcore_barrier()

        fd = [None, None]
        for k in range(_RPW // _FCH):
            p = k % 2
            if fd[p] is not None:
                fd[p].wait()
            r0 = s * _RPW + k * _FCH
            pltpu.sync_copy(acc.at[pl.ds(r0, _FCH)],
                            rows[p].at[pl.ds(0, _FCH)])
            fd[p] = pltpu.async_copy(rows[p].at[pl.ds(0, _FCH)],
                                     xout.at[pl.ds(coff + r0, _FCH)],
                                     gsems[p])
        for d in fd:
            d.wait()
        plsc.subcore_barrier()

    def _emit(ids, off0, out_ref):
        pltpu.sync_copy(ids.at[pl.ds(s * _UPW, _UPW)], uidv)
        for j in range(_UPW // 16):
            sl = pl.ds(j * 16, 16)
            base = uidv[sl] + off0
            uid0v[sl] = base * 2 + c
            uidv[sl] = base + coff
        dsts = ((uid0v, emb2, rows0, 0), (uidv, x1, rows0, _UPW),
                (uidv, x2, rows1, 0), (uidv, x3, rows1, _UPW))
        gd = [pltpu.async_copy(xt.at[iv], r.at[pl.ds(o, _UPW)], gsem0)
              for iv, xt, r, o in dsts]
        for d in gd:
            d.wait()

        def _avg(i, carry2):
            e0 = i * 8
            for u in range(8):
                e = e0 + u
                v = ((rows0[e, :] + rows0[_UPW + e, :])
                     + (rows1[e, :] + rows1[_UPW + e, :]))
                rows0[e, :] = v * jnp.float32(0.25)
            return carry2
        lax.fori_loop(0, _UPW // 8, _avg, 0)
        pltpu.sync_copy(rows0.at[pl.ds(0, _UPW)],
                        out_ref.at[pl.ds(c * _BATCH + s * _UPW, _UPW)])

    _emit(uids, 0, out_u)
    _emit(iids, _NUM_USERS, out_i)


@jax.jit
def _run(emb2, comb, uids, iids):
    f32, i32 = jnp.float32, jnp.int32
    call = pl.kernel(
        _body,
        out_type=[
            jax.ShapeDtypeStruct((2 * _N_PAD, _H), f32),
            jax.ShapeDtypeStruct((2 * _N_PAD, _H), f32),
            jax.ShapeDtypeStruct((2 * _N_PAD, _H), f32),
            jax.ShapeDtypeStruct((2 * _BATCH, _H), f32),
            jax.ShapeDtypeStruct((2 * _BATCH, _H), f32),
        ],
        mesh=plsc.VectorSubcoreMesh(core_axis_name="c", subcore_axis_name="s"),
        scratch_types=[
            pltpu.VMEM_SHARED((_N_PAD, _H), f32),
            pltpu.VMEM((3, _BE), i32),
            pltpu.VMEM((3, _BE), i32),
            pltpu.VMEM((3, _BE), i32),
            pltpu.VMEM((3, _BE), i32),
            pltpu.VMEM((_BE, _H), f32),
            pltpu.VMEM((_BE, _H), f32),
            pltpu.VMEM((_ZCH, _H), f32),
            pltpu.VMEM((_UPW,), i32),
            pltpu.VMEM((_UPW,), i32),
            pltpu.SemaphoreType.DMA,
            pltpu.SemaphoreType.DMA,
            pltpu.SemaphoreType.DMA,
            pltpu.SemaphoreType.DMA,
            pltpu.SemaphoreType.DMA,
        ],
        compiler_params=pltpu.CompilerParams(use_tc_tiling_on_sc=False,
                                             needs_layout_passes=False),
        name="light_gcn_sc",
    )
    return call(emb2, comb, uids, iids)


def kernel(user_emb, item_emb, edge_vals, edge_index, user_ids, item_ids):
    f32, i32 = jnp.float32, jnp.int32
    emb2 = jnp.concatenate([user_emb, item_emb], axis=0).reshape(2 * _N, _H)

    epad = _EPW * _NSUB - _E
    src_b = jnp.concatenate([edge_index[0],
                             jnp.zeros((epad,), i32)]).reshape(-1, _BE)
    dst_b = jnp.concatenate([edge_index[1],
                             jnp.zeros((epad,), i32)]).reshape(-1, _BE)
    val_b = jax.lax.bitcast_convert_type(
        jnp.concatenate([edge_vals, jnp.zeros((epad,), f32)]),
        i32).reshape(-1, _BE)
    comb = jnp.stack([src_b, dst_b, val_b], axis=1).reshape(-1, _BE)

    _, _, _, ou, oi = _run(emb2, comb, user_ids, item_ids)
    users = jnp.concatenate([ou[:_BATCH], ou[_BATCH:]], axis=1)
    items = jnp.concatenate([oi[:_BATCH], oi[_BATCH:]], axis=1)
    return users, items


# speedup vs baseline: 1.3737x; 1.0024x over previous
import jax
import jax.numpy as jnp
from jax import lax
from jax.experimental import pallas as pl
from jax.experimental.pallas import tpu as pltpu
from jax.experimental.pallas import tpu_sc as plsc

_NUM_USERS = 30000
_NUM_ITEMS = 70000
_N = _NUM_USERS + _NUM_ITEMS
_DIM = 32
_H = _DIM // 2
_E = 1600000
_HOP = 3
_BATCH = 4096

_NSUB = 16
_BE = 512
_NB = 200
_EPW = _BE * _NB
_NT = _NB // 2
_N_PAD = 100352
_RPW = _N_PAD // _NSUB
_ZCH = 224
_FCH = 448
_UPW = _BATCH // _NSUB


def _body(emb2, comb, uids, iids,
          x1, x2, x3, out_u, out_i,
          acc, ma0, mb0, ma1, mb1, rows0, rows1, zbuf, uidv, uid0v,
          gsem0, gsem1, lsem0, lsem1, zsem, ssem0, ssem1):
    c = lax.axis_index("c")
    s = lax.axis_index("s")
    coff = c * _N_PAD
    rows = (rows0, rows1)
    gsems = (gsem0, gsem1)
    lsems = (lsem0, lsem1)
    ssems = (ssem0, ssem1)
    metas = ((ma0, mb0), (ma1, mb1))

    def _zb(i, carry):
        zbuf[i, :] = jnp.zeros((_H,), jnp.float32)
        return carry
    lax.fori_loop(0, _ZCH, _zb, 0)

    def _meta_load(bidx, mref, sem):
        r0 = (s * _NB + bidx) * 3
        return pltpu.async_copy(comb.at[pl.ds(r0, 3)], mref, sem)

    def _xform(mref, h):
        for j in range(_BE // 16):
            sl = pl.ds(j * 16, 16)
            if h == 0:
                mref[0, sl] = mref[0, sl] * 2 + c
            else:
                mref[0, sl] = mref[0, sl] + coff

    def _wait_gather(p):
        pltpu.make_async_copy(emb2.at[pl.ds(0, _BE)], rows[p],
                              gsems[p]).wait()

    def _wait_meta(p):
        pltpu.make_async_copy(comb.at[pl.ds(0, 3)], metas[p][0],
                              lsems[p]).wait()

    tabs = [emb2, x1, x2, x3]
    for h in range(_HOP):
        xin, xout = tabs[h], tabs[h + 1]

        zd = [pltpu.async_copy(zbuf,
                               acc.at[pl.ds(s * _RPW + k * _ZCH, _ZCH)],
                               zsem)
              for k in range(_RPW // _ZCH)]
        for p in (0, 1):
            _meta_load(p, metas[p][0], lsems[p]).wait()
            _xform(metas[p][0], h)
            pltpu.async_copy(xin.at[metas[p][0].at[0]], rows[p], gsems[p])
            _meta_load(p + 2, metas[p][1], lsems[p])
        for d in zd:
            d.wait()
        plsc.subcore_barrier()

        def _blk(t2, carry):
            for q in (0, 1):
                t = 2 * t2 + q
                for p in (0, 1):
                    b = 2 * t + p
                    mx, my = metas[p][q], metas[p][1 - q]
                    _wait_gather(p)

                    def _mul(i, carry2):
                        e0 = i * 16
                        v16 = plsc.bitcast(mx[2, pl.ds(e0, 16)], jnp.float32)
                        for u in range(16):
                            e = e0 + u
                            rows[p][e, :] = rows[p][e, :] * v16[u]
                        return carry2
                    lax.fori_loop(0, _BE // 16, _mul, 0)

                    sd = pltpu.async_copy(rows[p], acc.at[mx.at[1]],
                                          ssems[p], add=True)
                    _wait_meta(p)
                    _xform(my, h)
                    sd.wait()
                    _meta_load(jnp.minimum(b + 4, _NB - 1), mx, lsems[p])
                    pltpu.async_copy(xin.at[my.at[0]], rows[p], gsems[p])
            return carry
        lax.fori_loop(0, _NT // 2, _blk, 0)
        for p in (0, 1):
            _wait_gather(p)
            _wait_meta(p)
        plsc.subcore_barrier()

        fd = [None, None]
        for k in range(_RPW // _FCH):
            p = k % 2
            if fd[p] is not None:
                fd[p].wait()
            r0 = s * _RPW + k * _FCH
            pltpu.sync_copy(acc.at[pl.ds(r0, _FCH)],
                            rows[p].at[pl.ds(0, _FCH)])
            fd[p] = pltpu.async_copy(rows[p].at[pl.ds(0, _FCH)],
                                     xout.at[pl.ds(coff + r0, _FCH)],
                                     gsems[p])
        for d in fd:
            d.wait()
        plsc.subcore_barrier()

    def _emit(ids, off0, out_ref):
        pltpu.sync_copy(ids.at[pl.ds(s * _UPW, _UPW)], uidv)
        for j in range(_UPW // 16):
            sl = pl.ds(j * 16, 16)
            base = uidv[sl] + off0
            uid0v[sl] = base * 2 + c
            uidv[sl] = base + coff
        dsts = ((uid0v, emb2, rows0, 0), (uidv, x1, rows0, _UPW),
                (uidv, x2, rows1, 0), (uidv, x3, rows1, _UPW))
        gd = [pltpu.async_copy(xt.at[iv], r.at[pl.ds(o, _UPW)], gsem0)
              for iv, xt, r, o in dsts]
        for d in gd:
            d.wait()

        def _avg(i, carry2):
            e0 = i * 8
            for u in range(8):
                e = e0 + u
                v = ((rows0[e, :] + rows0[_UPW + e, :])
                     + (rows1[e, :] + rows1[_UPW + e, :]))
                rows0[e, :] = v * jnp.float32(0.25)
            return carry2
        lax.fori_loop(0, _UPW // 8, _avg, 0)
        pltpu.sync_copy(rows0.at[pl.ds(0, _UPW)],
                        out_ref.at[pl.ds(c * _BATCH + s * _UPW, _UPW)])

    _emit(uids, 0, out_u)
    _emit(iids, _NUM_USERS, out_i)


@jax.jit
def _run(emb2, comb, uids, iids):
    f32, i32 = jnp.float32, jnp.int32
    call = pl.kernel(
        _body,
        out_type=[
            jax.ShapeDtypeStruct((2 * _N_PAD, _H), f32),
            jax.ShapeDtypeStruct((2 * _N_PAD, _H), f32),
            jax.ShapeDtypeStruct((2 * _N_PAD, _H), f32),
            jax.ShapeDtypeStruct((2 * _BATCH, _H), f32),
            jax.ShapeDtypeStruct((2 * _BATCH, _H), f32),
        ],
        mesh=plsc.VectorSubcoreMesh(core_axis_name="c", subcore_axis_name="s"),
        scratch_types=[
            pltpu.VMEM_SHARED((_N_PAD, _H), f32),
            pltpu.VMEM((3, _BE), i32),
            pltpu.VMEM((3, _BE), i32),
            pltpu.VMEM((3, _BE), i32),
            pltpu.VMEM((3, _BE), i32),
            pltpu.VMEM((_BE, _H), f32),
            pltpu.VMEM((_BE, _H), f32),
            pltpu.VMEM((_ZCH, _H), f32),
            pltpu.VMEM((_UPW,), i32),
            pltpu.VMEM((_UPW,), i32),
            pltpu.SemaphoreType.DMA,
            pltpu.SemaphoreType.DMA,
            pltpu.SemaphoreType.DMA,
            pltpu.SemaphoreType.DMA,
            pltpu.SemaphoreType.DMA,
            pltpu.SemaphoreType.DMA,
            pltpu.SemaphoreType.DMA,
        ],
        compiler_params=pltpu.CompilerParams(use_tc_tiling_on_sc=False,
                                             needs_layout_passes=False),
        name="light_gcn_sc",
    )
    return call(emb2, comb, uids, iids)


def kernel(user_emb, item_emb, edge_vals, edge_index, user_ids, item_ids):
    f32, i32 = jnp.float32, jnp.int32
    emb2 = jnp.concatenate([user_emb, item_emb], axis=0).reshape(2 * _N, _H)

    epad = _EPW * _NSUB - _E
    src_b = jnp.concatenate([edge_index[0],
                             jnp.zeros((epad,), i32)]).reshape(-1, _BE)
    dst_b = jnp.concatenate([edge_index[1],
                             jnp.zeros((epad,), i32)]).reshape(-1, _BE)
    val_b = jax.lax.bitcast_convert_type(
        jnp.concatenate([edge_vals, jnp.zeros((epad,), f32)]),
        i32).reshape(-1, _BE)
    comb = jnp.stack([src_b, dst_b, val_b], axis=1).reshape(-1, _BE)

    _, _, _, ou, oi = _run(emb2, comb, user_ids, item_ids)
    users = jnp.concatenate([ou[:_BATCH], ou[_BATCH:]], axis=1)
    items = jnp.concatenate([oi[:_BATCH], oi[_BATCH:]], axis=1)
    return users, items
